# baseline (speedup 1.0000x reference)
import functools
import numpy as np
import jax
import jax.numpy as jnp
from jax.experimental import pallas as pl
from jax.experimental.pallas import tpu as pltpu
from jax.experimental.layout import Format, Layout

_C, _D, _H, _W = 4, 128, 224, 224
_NS = 64
_DBLK = 32
_NK = _D // _DBLK
_SBLK = _NS // _NK

_AX = np.linspace(0, _D - 1, _NS).astype(np.int32)
_SG = np.linspace(0, _W - 1, _NS).astype(np.int32)
_CO = np.linspace(0, _H - 1, _NS).astype(np.int32)

assert all(_AX[k * _SBLK + j] // _DBLK == k
           for k in range(_NK) for j in range(_SBLK))
_AX_LOCAL = _AX.reshape(_NK, _SBLK) - (np.arange(_NK) * _DBLK)[:, None]


def _onehot(idx, n):
    m = np.zeros((_NS, n), np.float32)
    m[np.arange(_NS), idx] = 1.0
    return jnp.asarray(m)


def _body(oh_co_ref, oh_sg_ref, vol_ref, ax_ref, sag_ref, cor_ref, tp_ref):
    k = pl.program_id(1)
    for p in range(_DBLK):
        tp_ref[p] = vol_ref[0, p].T
    for s in range(_NS):
        sag_ref[0, s, :, :] = tp_ref[:, int(_SG[s]), :]
    for s in range(_NS):
        cor_ref[0, s, :, :] = vol_ref[0, :, int(_CO[s]), :]
    for j in range(_SBLK):
        if np.all(_AX_LOCAL[:, j] == _AX_LOCAL[0, j]):
            ax_ref[0, j] = vol_ref[0, int(_AX_LOCAL[0, j])]
        else:
            lj = jnp.where(k == _NK - 1, int(_AX_LOCAL[-1, j]),
                           int(_AX_LOCAL[0, j]))
            ax_ref[0, j] = vol_ref[0, lj]


def _fmt(ndim):
    dev = jax.devices()[0]
    return Format(Layout(tuple(range(ndim)), ((8, 128),)),
                  jax.sharding.SingleDeviceSharding(dev))


@functools.lru_cache(maxsize=1)
def _jitted():
    return jax.jit(_impl, out_shardings=(_fmt(4), _fmt(4), _fmt(4)))


def kernel(volume):
    return _jitted()(volume)


def _impl(volume):
    oh_co = _onehot(_CO, _H)
    oh_sg = _onehot(_SG, _W)
    grid = (_C, _NK)
    out = pl.pallas_call(
        _body,
        grid=grid,
        in_specs=[
            pl.BlockSpec((_NS, _H), lambda c, k: (0, 0)),
            pl.BlockSpec((_NS, _W), lambda c, k: (0, 0)),
            pl.BlockSpec((1, _DBLK, _H, _W), lambda c, k: (c, k, 0, 0)),
        ],
        out_specs=[
            pl.BlockSpec((1, _SBLK, _H, _W), lambda c, k: (c, k, 0, 0)),
            pl.BlockSpec((1, _NS, _DBLK, _H), lambda c, k: (c, 0, k, 0)),
            pl.BlockSpec((1, _NS, _DBLK, _W), lambda c, k: (c, 0, k, 0)),
        ],
        out_shape=[
            jax.ShapeDtypeStruct((_C, _NS, _H, _W), jnp.float32),
            jax.ShapeDtypeStruct((_C, _NS, _D, _H), jnp.float32),
            jax.ShapeDtypeStruct((_C, _NS, _D, _W), jnp.float32),
        ],
        scratch_shapes=[pltpu.VMEM((_DBLK, _W, _H), jnp.float32)],
        compiler_params=pltpu.CompilerParams(
            dimension_semantics=("parallel", "parallel")),
    )(oh_co, oh_sg, volume)
    axial, sagittal, coronal = out
    return (axial, sagittal, coronal)


# speedup vs baseline: 3.1979x; 3.1979x over previous
import numpy as np
import jax
import jax.numpy as jnp
from jax.experimental import pallas as pl
from jax.experimental.pallas import tpu as pltpu

_C, _D, _H, _W = 4, 128, 224, 224
_NS = 64
_HB = 56
_NKH = _H // _HB
_CPB = _NS // _NKH

_AX = np.linspace(0, _D - 1, _NS).astype(np.int32)
_SG = np.linspace(0, _W - 1, _NS).astype(np.int32)
_CO = np.linspace(0, _H - 1, _NS).astype(np.int32)
assert all(int(_SG[s]) == (s * (_W - 1)) // (_NS - 1) for s in range(_NS))
assert all(int(_AX[s]) == (2 * s if s < 63 else 127) for s in range(_NS))
assert all(_CO[k * _CPB + j] // _HB == k
           for k in range(_NKH) for j in range(_CPB))


def _body(vt_ref, ax_ref, sagp_ref, corp_ref, tps_ref):
    pk = pl.program_id(1)
    for j in range(_CPB):
        s = pk * _CPB + j
        h_loc = jax.lax.div(s * (_H - 1), _NS - 1) - pk * _HB
        corp_ref[0, j] = vt_ref[0, h_loc]
    for s in range(_NS):
        sagp_ref[0, s, :, :] = vt_ref[0, :, int(_SG[s]), :]
    for hp in range(_HB):
        tps_ref[hp] = vt_ref[0, hp].T
    for s in range(_NS):
        ax_ref[0, s, :, :] = tps_ref[:, int(_AX[s]), :]


@jax.jit
def kernel(volume):
    vt = jnp.transpose(volume, (0, 2, 3, 1))
    ax, sagp, corp = pl.pallas_call(
        _body,
        grid=(_C, _NKH),
        in_specs=[
            pl.BlockSpec((1, _HB, _W, _D), lambda c, k: (c, k, 0, 0)),
        ],
        out_specs=[
            pl.BlockSpec((1, _NS, _HB, _W), lambda c, k: (c, 0, k, 0)),
            pl.BlockSpec((1, _NS, _HB, _D), lambda c, k: (c, 0, k, 0)),
            pl.BlockSpec((1, _CPB, _W, _D), lambda c, k: (c, k, 0, 0)),
        ],
        out_shape=[
            jax.ShapeDtypeStruct((_C, _NS, _H, _W), jnp.float32),
            jax.ShapeDtypeStruct((_C, _NS, _H, _D), jnp.float32),
            jax.ShapeDtypeStruct((_C, _NS, _W, _D), jnp.float32),
        ],
        scratch_shapes=[pltpu.VMEM((_HB, _D, _W), jnp.float32)],
        compiler_params=pltpu.CompilerParams(
            dimension_semantics=("parallel", "parallel")),
    )(vt)
    sagittal = jnp.transpose(sagp, (0, 1, 3, 2))
    coronal = jnp.transpose(corp, (0, 1, 3, 2))
    return (ax, sagittal, coronal)
